# baseline (speedup 1.0000x reference)
import jax
import jax.numpy as jnp
from jax import lax
from jax.experimental import pallas as pl
from jax.experimental.pallas import tpu as pltpu
from jax.experimental.pallas import tpu_sc as plsc

_B, _D, _H, _W = 8, 48, 160, 320
_ROWS = 8
_CPX = _ROWS * _W
_NCHUNK = 5
_DQ = 12
_NQ = _D // _DQ
_LANES = 16
_GROUPS = _CPX // _LANES
_WPB = 4


def _body(cost_hbm, out_hbm, buf, st, obuf, isem0, isem1, osem):
    nc = 2
    wid = lax.axis_index("s") * nc + lax.axis_index("c")
    b = wid // _WPB
    row0 = (wid % _WPB) * (_NCHUNK * _ROWS)
    isems = (isem0, isem1)

    def in_copy(c, q, slot):
        src = cost_hbm.at[b, pl.ds(q * _DQ, _DQ), pl.ds(row0 + c * _ROWS, _ROWS), :]
        return pltpu.make_async_copy(src, buf.at[slot], isems[slot])

    def out_copy(c):
        dst = out_hbm.at[b, 0, pl.ds(row0 + c * _ROWS, _ROWS), :]
        return pltpu.make_async_copy(obuf.at[pl.ds(c * _ROWS, _ROWS), :], dst, osem)

    def compute_tile(q, slot, c):
        def body(ww, _):
            w16 = ww * _LANES
            for hh in range(_ROWS):
                s16 = hh * _W + w16
                if q == 0:
                    v = buf[slot, 0, hh, pl.ds(w16, _LANES)]
                    max1 = v
                    i1 = jnp.zeros((_LANES,), jnp.float32)
                    max2 = jnp.full((_LANES,), -jnp.inf, jnp.float32)
                    i2 = jnp.zeros((_LANES,), jnp.float32)
                    dds = range(1, _DQ)
                else:
                    max1 = st[0, pl.ds(s16, _LANES)]
                    max2 = st[1, pl.ds(s16, _LANES)]
                    i1 = st[2, pl.ds(s16, _LANES)]
                    i2 = st[3, pl.ds(s16, _LANES)]
                    dds = range(_DQ)
                for dd in dds:
                    v = buf[slot, dd, hh, pl.ds(w16, _LANES)]
                    df = jnp.float32(q * _DQ + dd)
                    gt1 = v > max1
                    gt2 = v > max2
                    i2 = jnp.where(gt1, i1, jnp.where(gt2, df, i2))
                    max2 = jnp.where(gt1, max1, jnp.where(gt2, v, max2))
                    i1 = jnp.where(gt1, df, i1)
                    max1 = jnp.where(gt1, v, max1)
                if q == _NQ - 1:
                    e = jnp.exp(max2 - max1)
                    obuf[c * _ROWS + hh, pl.ds(w16, _LANES)] = (
                        i1 + i2 * e
                    ) / (1.0 + e)
                else:
                    st[0, pl.ds(s16, _LANES)] = max1
                    st[1, pl.ds(s16, _LANES)] = max2
                    st[2, pl.ds(s16, _LANES)] = i1
                    st[3, pl.ds(s16, _LANES)] = i2
            return 0

        lax.fori_loop(0, _W // _LANES, body, 0)

    in_copy(0, 0, 0).start()

    def chunk(c, _):
        for q in range(_NQ):
            slot = q % 2
            in_copy(c, q, slot).wait()
            if q + 1 < _NQ:
                in_copy(c, q + 1, 1 - slot).start()
            else:

                @pl.when(c + 1 < _NCHUNK)
                def _():
                    in_copy(c + 1, 0, 1 - slot).start()

            compute_tile(q, slot, c)
        out_copy(c).start()
        return 0

    lax.fori_loop(0, _NCHUNK, chunk, 0)
    for _ in range(_NCHUNK):
        out_copy(0).wait()


@jax.jit
def kernel(cost):
    mesh = plsc.VectorSubcoreMesh(
        core_axis_name="c", subcore_axis_name="s", num_cores=2, num_subcores=16
    )
    return pl.kernel(
        _body,
        out_type=jax.ShapeDtypeStruct((_B, 1, _H, _W), jnp.float32),
        mesh=mesh,
        scratch_types=[
            pltpu.VMEM((2, _DQ, _ROWS, _W), jnp.float32),
            pltpu.VMEM((4, _CPX), jnp.float32),
            pltpu.VMEM((_NCHUNK * _ROWS, _W), jnp.float32),
            pltpu.SemaphoreType.DMA,
            pltpu.SemaphoreType.DMA,
            pltpu.SemaphoreType.DMA,
        ],
    )(cost)


# speedup vs baseline: 1.9291x; 1.9291x over previous
import jax
import jax.numpy as jnp
from jax import lax
from jax.experimental import pallas as pl
from jax.experimental.pallas import tpu as pltpu
from jax.experimental.pallas import tpu_sc as plsc

_B, _D, _H, _W = 8, 48, 160, 320
_SC_H = 32
_TC_H = _H - _SC_H
_ROWS = 8
_DQ = 12
_NQ = _D // _DQ
_LANES = 16
_WPB = _SC_H // _ROWS
_TC_BH = 32


def _top2_update(v, df, max1, max2, i1, i2):
    gt1 = v > max1
    gt2 = v > max2
    i2 = jnp.where(gt1, i1, jnp.where(gt2, df, i2))
    max2 = jnp.where(gt1, max1, jnp.where(gt2, v, max2))
    i1 = jnp.where(gt1, df, i1)
    max1 = jnp.where(gt1, v, max1)
    return max1, max2, i1, i2


def _sc_body(cost_hbm, out_hbm, buf, st, obuf, isem0, isem1, osem):
    nc = 2
    wid = lax.axis_index("s") * nc + lax.axis_index("c")
    b = wid // _WPB
    row0 = (wid % _WPB) * _ROWS
    isems = (isem0, isem1)

    def in_copy(q, slot):
        src = cost_hbm.at[b, pl.ds(q * _DQ, _DQ), pl.ds(row0, _ROWS), :]
        return pltpu.make_async_copy(src, buf.at[slot], isems[slot])

    def compute_tile(q, slot):
        def body(ww, _):
            w16 = ww * _LANES
            for hh in range(_ROWS):
                s16 = hh * _W + w16
                if q == 0:
                    v = buf[slot, 0, hh, pl.ds(w16, _LANES)]
                    max1 = v
                    i1 = jnp.zeros((_LANES,), jnp.float32)
                    max2 = jnp.full((_LANES,), -jnp.inf, jnp.float32)
                    i2 = jnp.zeros((_LANES,), jnp.float32)
                    dds = range(1, _DQ)
                else:
                    max1 = st[0, pl.ds(s16, _LANES)]
                    max2 = st[1, pl.ds(s16, _LANES)]
                    i1 = st[2, pl.ds(s16, _LANES)]
                    i2 = st[3, pl.ds(s16, _LANES)]
                    dds = range(_DQ)
                for dd in dds:
                    v = buf[slot, dd, hh, pl.ds(w16, _LANES)]
                    df = jnp.float32(q * _DQ + dd)
                    max1, max2, i1, i2 = _top2_update(v, df, max1, max2, i1, i2)
                if q == _NQ - 1:
                    e = jnp.exp(max2 - max1)
                    obuf[hh, pl.ds(w16, _LANES)] = (i1 + i2 * e) / (1.0 + e)
                else:
                    st[0, pl.ds(s16, _LANES)] = max1
                    st[1, pl.ds(s16, _LANES)] = max2
                    st[2, pl.ds(s16, _LANES)] = i1
                    st[3, pl.ds(s16, _LANES)] = i2
            return 0

        lax.fori_loop(0, _W // _LANES, body, 0)

    in_copy(0, 0).start()
    for q in range(_NQ):
        slot = q % 2
        in_copy(q, slot).wait()
        if q + 1 < _NQ:
            in_copy(q + 1, 1 - slot).start()
        compute_tile(q, slot)
    dst = out_hbm.at[b, 0, pl.ds(row0, _ROWS), :]
    pltpu.make_async_copy(obuf, dst, osem).start()
    pltpu.make_async_copy(obuf, dst, osem).wait()


def _tc_body(x_ref, o_ref):
    x = x_ref[0]
    max1 = x[0]
    shape = max1.shape
    i1 = jnp.zeros(shape, jnp.float32)
    max2 = jnp.full(shape, -jnp.inf, jnp.float32)
    i2 = jnp.zeros(shape, jnp.float32)
    for d in range(1, _D):
        max1, max2, i1, i2 = _top2_update(
            x[d], jnp.float32(d), max1, max2, i1, i2
        )
    e = jnp.exp(max2 - max1)
    o_ref[0, 0] = (i1 + i2 * e) / (1.0 + e)


@jax.jit
def kernel(cost):
    mesh = plsc.VectorSubcoreMesh(
        core_axis_name="c", subcore_axis_name="s", num_cores=2, num_subcores=16
    )
    disp_sc = pl.kernel(
        _sc_body,
        out_type=jax.ShapeDtypeStruct((_B, 1, _SC_H, _W), jnp.float32),
        mesh=mesh,
        scratch_types=[
            pltpu.VMEM((2, _DQ, _ROWS, _W), jnp.float32),
            pltpu.VMEM((4, _ROWS * _W), jnp.float32),
            pltpu.VMEM((_ROWS, _W), jnp.float32),
            pltpu.SemaphoreType.DMA,
            pltpu.SemaphoreType.DMA,
            pltpu.SemaphoreType.DMA,
        ],
    )(cost)

    disp_tc = pl.pallas_call(
        _tc_body,
        grid=(_B, _TC_H // _TC_BH),
        in_specs=[
            pl.BlockSpec(
                (1, _D, _TC_BH, _W),
                lambda i, j: (i, 0, j + _SC_H // _TC_BH, 0),
            )
        ],
        out_specs=pl.BlockSpec((1, 1, _TC_BH, _W), lambda i, j: (i, 0, j, 0)),
        out_shape=jax.ShapeDtypeStruct((_B, 1, _TC_H, _W), jnp.float32),
    )(cost)

    return jnp.concatenate([disp_sc, disp_tc], axis=2)


# speedup vs baseline: 2.1890x; 1.1347x over previous
import jax
import jax.numpy as jnp
from jax import lax
from jax.experimental import pallas as pl
from jax.experimental.pallas import tpu as pltpu
from jax.experimental.pallas import tpu_sc as plsc

_B, _D, _H, _W = 8, 48, 160, 320
_SC_H = 32
_TC_H = _H - _SC_H
_ROWS = 8
_DQ = 12
_NQ = _D // _DQ
_LANES = 16
_WPB = _SC_H // _ROWS
_TC_BH = 64


def _top2_update(v, df, max1, max2, i1, i2):
    gt1 = v > max1
    gt2 = v > max2
    i2 = jnp.where(gt1, i1, jnp.where(gt2, df, i2))
    max2 = jnp.where(gt1, max1, jnp.where(gt2, v, max2))
    i1 = jnp.where(gt1, df, i1)
    max1 = jnp.where(gt1, v, max1)
    return max1, max2, i1, i2


def _sc_body(cost_hbm, out_hbm, buf, st, obuf, isem0, isem1, osem):
    nc = 2
    wid = lax.axis_index("s") * nc + lax.axis_index("c")
    b = wid // _WPB
    orow0 = (wid % _WPB) * _ROWS
    row0 = _TC_H + orow0
    isems = (isem0, isem1)

    def in_copy(q, slot):
        src = cost_hbm.at[b, pl.ds(q * _DQ, _DQ), pl.ds(row0, _ROWS), :]
        return pltpu.make_async_copy(src, buf.at[slot], isems[slot])

    def compute_tile(q, slot):
        def body(ww, _):
            w16 = ww * _LANES
            for hh in range(_ROWS):
                s16 = hh * _W + w16
                if q == 0:
                    v = buf[slot, 0, hh, pl.ds(w16, _LANES)]
                    max1 = v
                    i1 = jnp.zeros((_LANES,), jnp.float32)
                    max2 = jnp.full((_LANES,), -jnp.inf, jnp.float32)
                    i2 = jnp.zeros((_LANES,), jnp.float32)
                    dds = range(1, _DQ)
                else:
                    max1 = st[0, pl.ds(s16, _LANES)]
                    max2 = st[1, pl.ds(s16, _LANES)]
                    i1 = st[2, pl.ds(s16, _LANES)]
                    i2 = st[3, pl.ds(s16, _LANES)]
                    dds = range(_DQ)
                for dd in dds:
                    v = buf[slot, dd, hh, pl.ds(w16, _LANES)]
                    df = jnp.float32(q * _DQ + dd)
                    max1, max2, i1, i2 = _top2_update(v, df, max1, max2, i1, i2)
                if q == _NQ - 1:
                    e = jnp.exp(max2 - max1)
                    obuf[hh, pl.ds(w16, _LANES)] = (i1 + i2 * e) / (1.0 + e)
                else:
                    st[0, pl.ds(s16, _LANES)] = max1
                    st[1, pl.ds(s16, _LANES)] = max2
                    st[2, pl.ds(s16, _LANES)] = i1
                    st[3, pl.ds(s16, _LANES)] = i2
            return 0

        lax.fori_loop(0, _W // _LANES, body, 0)

    in_copy(0, 0).start()
    for q in range(_NQ):
        slot = q % 2
        in_copy(q, slot).wait()
        if q + 1 < _NQ:
            in_copy(q + 1, 1 - slot).start()
        compute_tile(q, slot)
    dst = out_hbm.at[b, 0, pl.ds(orow0, _ROWS), :]
    pltpu.make_async_copy(obuf, dst, osem).start()
    pltpu.make_async_copy(obuf, dst, osem).wait()


def _tc_body(x_ref, o_ref):
    x = x_ref[0]
    max1 = x[0]
    shape = max1.shape
    i1 = jnp.zeros(shape, jnp.float32)
    max2 = jnp.full(shape, -jnp.inf, jnp.float32)
    i2 = jnp.zeros(shape, jnp.float32)
    for d in range(1, _D):
        max1, max2, i1, i2 = _top2_update(
            x[d], jnp.float32(d), max1, max2, i1, i2
        )
    e = jnp.exp(max2 - max1)
    o_ref[0, 0] = (i1 + i2 * e) / (1.0 + e)


@jax.jit
def kernel(cost):
    mesh = plsc.VectorSubcoreMesh(
        core_axis_name="c", subcore_axis_name="s", num_cores=2, num_subcores=16
    )
    disp_sc = pl.kernel(
        _sc_body,
        out_type=jax.ShapeDtypeStruct((_B, 1, _SC_H, _W), jnp.float32),
        mesh=mesh,
        scratch_types=[
            pltpu.VMEM((2, _DQ, _ROWS, _W), jnp.float32),
            pltpu.VMEM((4, _ROWS * _W), jnp.float32),
            pltpu.VMEM((_ROWS, _W), jnp.float32),
            pltpu.SemaphoreType.DMA,
            pltpu.SemaphoreType.DMA,
            pltpu.SemaphoreType.DMA,
        ],
    )(cost)

    disp_tc = pl.pallas_call(
        _tc_body,
        grid=(_B, _TC_H // _TC_BH),
        in_specs=[
            pl.BlockSpec((1, _D, _TC_BH, _W), lambda i, j: (i, 0, j, 0))
        ],
        out_specs=pl.BlockSpec((1, 1, _TC_BH, _W), lambda i, j: (i, 0, j, 0)),
        out_shape=jax.ShapeDtypeStruct((_B, 1, _TC_H, _W), jnp.float32),
    )(cost)

    return jnp.concatenate([disp_tc, disp_sc], axis=2)


# speedup vs baseline: 2.2856x; 1.0441x over previous
import jax
import jax.numpy as jnp
from jax import lax
from jax.experimental import pallas as pl
from jax.experimental.pallas import tpu as pltpu
from jax.experimental.pallas import tpu_sc as plsc

_B, _D, _H, _W = 8, 48, 160, 320
_SC_H = 32
_TC_H = _H - _SC_H
_ROWS = 8
_DQ = 12
_NQ = _D // _DQ
_LANES = 16
_WPB = _SC_H // _ROWS
_TC_BH = 128


def _top2_update(v, df, max1, max2, i1, i2):
    gt1 = v > max1
    gt2 = v > max2
    i2 = jnp.where(gt1, i1, jnp.where(gt2, df, i2))
    max2 = jnp.where(gt1, max1, jnp.where(gt2, v, max2))
    i1 = jnp.where(gt1, df, i1)
    max1 = jnp.where(gt1, v, max1)
    return max1, max2, i1, i2


def _sc_body(cost_hbm, out_hbm, buf, st, obuf, isem0, isem1, osem):
    nc = 2
    wid = lax.axis_index("s") * nc + lax.axis_index("c")
    b = wid // _WPB
    orow0 = (wid % _WPB) * _ROWS
    row0 = _TC_H + orow0
    isems = (isem0, isem1)

    def in_copy(q, slot):
        src = cost_hbm.at[b, pl.ds(q * _DQ, _DQ), pl.ds(row0, _ROWS), :]
        return pltpu.make_async_copy(src, buf.at[slot], isems[slot])

    def compute_tile(q, slot):
        def body(ww, _):
            w16 = ww * _LANES
            for hh in range(_ROWS):
                s16 = hh * _W + w16
                if q == 0:
                    v = buf[slot, 0, hh, pl.ds(w16, _LANES)]
                    max1 = v
                    i1 = jnp.zeros((_LANES,), jnp.float32)
                    max2 = jnp.full((_LANES,), -jnp.inf, jnp.float32)
                    i2 = jnp.zeros((_LANES,), jnp.float32)
                    dds = range(1, _DQ)
                else:
                    max1 = st[0, pl.ds(s16, _LANES)]
                    max2 = st[1, pl.ds(s16, _LANES)]
                    i1 = st[2, pl.ds(s16, _LANES)]
                    i2 = st[3, pl.ds(s16, _LANES)]
                    dds = range(_DQ)
                for dd in dds:
                    v = buf[slot, dd, hh, pl.ds(w16, _LANES)]
                    df = jnp.float32(q * _DQ + dd)
                    max1, max2, i1, i2 = _top2_update(v, df, max1, max2, i1, i2)
                if q == _NQ - 1:
                    e = jnp.exp(max2 - max1)
                    obuf[hh, pl.ds(w16, _LANES)] = (i1 + i2 * e) / (1.0 + e)
                else:
                    st[0, pl.ds(s16, _LANES)] = max1
                    st[1, pl.ds(s16, _LANES)] = max2
                    st[2, pl.ds(s16, _LANES)] = i1
                    st[3, pl.ds(s16, _LANES)] = i2
            return 0

        lax.fori_loop(0, _W // _LANES, body, 0)

    in_copy(0, 0).start()
    for q in range(_NQ):
        slot = q % 2
        in_copy(q, slot).wait()
        if q + 1 < _NQ:
            in_copy(q + 1, 1 - slot).start()
        compute_tile(q, slot)
    dst = out_hbm.at[b, 0, pl.ds(orow0, _ROWS), :]
    pltpu.make_async_copy(obuf, dst, osem).start()
    pltpu.make_async_copy(obuf, dst, osem).wait()


def _tc_body(x_ref, o_ref):
    x = x_ref[0]
    max1 = x[0]
    shape = max1.shape
    i1 = jnp.zeros(shape, jnp.float32)
    max2 = jnp.full(shape, -jnp.inf, jnp.float32)
    i2 = jnp.zeros(shape, jnp.float32)
    for d in range(1, _D):
        max1, max2, i1, i2 = _top2_update(
            x[d], jnp.float32(d), max1, max2, i1, i2
        )
    e = jnp.exp(max2 - max1)
    o_ref[0, 0] = (i1 + i2 * e) / (1.0 + e)


@jax.jit
def kernel(cost):
    mesh = plsc.VectorSubcoreMesh(
        core_axis_name="c", subcore_axis_name="s", num_cores=2, num_subcores=16
    )
    disp_sc = pl.kernel(
        _sc_body,
        out_type=jax.ShapeDtypeStruct((_B, 1, _SC_H, _W), jnp.float32),
        mesh=mesh,
        scratch_types=[
            pltpu.VMEM((2, _DQ, _ROWS, _W), jnp.float32),
            pltpu.VMEM((4, _ROWS * _W), jnp.float32),
            pltpu.VMEM((_ROWS, _W), jnp.float32),
            pltpu.SemaphoreType.DMA,
            pltpu.SemaphoreType.DMA,
            pltpu.SemaphoreType.DMA,
        ],
    )(cost)

    disp_tc = pl.pallas_call(
        _tc_body,
        grid=(_B, _TC_H // _TC_BH),
        in_specs=[
            pl.BlockSpec((1, _D, _TC_BH, _W), lambda i, j: (i, 0, j, 0))
        ],
        out_specs=pl.BlockSpec((1, 1, _TC_BH, _W), lambda i, j: (i, 0, j, 0)),
        out_shape=jax.ShapeDtypeStruct((_B, 1, _TC_H, _W), jnp.float32),
    )(cost)

    return jnp.concatenate([disp_tc, disp_sc], axis=2)


# speedup vs baseline: 2.3388x; 1.0233x over previous
import jax
import jax.numpy as jnp
from jax import lax
from jax.experimental import pallas as pl
from jax.experimental.pallas import tpu as pltpu
from jax.experimental.pallas import tpu_sc as plsc

_B, _D, _H, _W = 8, 48, 160, 320
_SC_H = 32
_TC_H = _H - _SC_H
_ROWS = 8
_DQ = 12
_NQ = _D // _DQ
_LANES = 16
_WPB = _SC_H // _ROWS
_TC_BH = 128


def _top2_update(v, df, max1, max2, i1, i2):
    gt1 = v > max1
    gt2 = v > max2
    i2 = jnp.where(gt1, i1, jnp.where(gt2, df, i2))
    max2 = jnp.where(gt1, max1, jnp.where(gt2, v, max2))
    i1 = jnp.where(gt1, df, i1)
    max1 = jnp.where(gt1, v, max1)
    return max1, max2, i1, i2


def _sc_body(cost_hbm, out_hbm, buf, st, obuf, isem0, isem1, osem):
    nc = 2
    wid = lax.axis_index("s") * nc + lax.axis_index("c")
    b = wid // _WPB
    orow0 = (wid % _WPB) * _ROWS
    row0 = _TC_H + orow0
    isems = (isem0, isem1)

    def in_copy(q, slot):
        src = cost_hbm.at[b, pl.ds(q * _DQ, _DQ), pl.ds(row0, _ROWS), :]
        return pltpu.make_async_copy(src, buf.at[slot], isems[slot])

    def compute_tile(q, slot):
        def body(ww, _):
            w16 = ww * _LANES
            for hh in range(_ROWS):
                s16 = hh * _W + w16
                if q == 0:
                    v = buf[slot, 0, hh, pl.ds(w16, _LANES)]
                    max1 = v
                    i1 = jnp.zeros((_LANES,), jnp.float32)
                    max2 = jnp.full((_LANES,), -jnp.inf, jnp.float32)
                    i2 = jnp.zeros((_LANES,), jnp.float32)
                    dds = range(1, _DQ)
                else:
                    max1 = st[0, pl.ds(s16, _LANES)]
                    max2 = st[1, pl.ds(s16, _LANES)]
                    i1 = st[2, pl.ds(s16, _LANES)]
                    i2 = st[3, pl.ds(s16, _LANES)]
                    dds = range(_DQ)
                for dd in dds:
                    v = buf[slot, dd, hh, pl.ds(w16, _LANES)]
                    df = jnp.float32(q * _DQ + dd)
                    max1, max2, i1, i2 = _top2_update(v, df, max1, max2, i1, i2)
                if q == _NQ - 1:
                    e = jnp.exp(max2 - max1)
                    obuf[hh, pl.ds(w16, _LANES)] = (i1 + i2 * e) / (1.0 + e)
                else:
                    st[0, pl.ds(s16, _LANES)] = max1
                    st[1, pl.ds(s16, _LANES)] = max2
                    st[2, pl.ds(s16, _LANES)] = i1
                    st[3, pl.ds(s16, _LANES)] = i2
            return 0

        lax.fori_loop(0, _W // _LANES, body, 0)

    in_copy(0, 0).start()
    for q in range(_NQ):
        slot = q % 2
        in_copy(q, slot).wait()
        if q + 1 < _NQ:
            in_copy(q + 1, 1 - slot).start()
        compute_tile(q, slot)
    dst = out_hbm.at[b, 0, pl.ds(orow0, _ROWS), :]
    pltpu.make_async_copy(obuf, dst, osem).start()
    pltpu.make_async_copy(obuf, dst, osem).wait()


def _tc_body(x_ref, o_ref):
    x = x_ref[0]
    max1 = x[0]
    shape = max1.shape
    i1 = jnp.zeros(shape, jnp.float32)
    max2 = jnp.full(shape, -jnp.inf, jnp.float32)
    i2 = jnp.zeros(shape, jnp.float32)
    for d in range(1, _D):
        max1, max2, i1, i2 = _top2_update(
            x[d], jnp.float32(d), max1, max2, i1, i2
        )
    e = jnp.exp(max2 - max1)
    o_ref[0, 0] = (i1 + i2 * e) / (1.0 + e)


@jax.jit
def kernel(cost):
    mesh = plsc.VectorSubcoreMesh(
        core_axis_name="c", subcore_axis_name="s", num_cores=2, num_subcores=16
    )
    disp_sc = pl.kernel(
        _sc_body,
        out_type=jax.ShapeDtypeStruct((_B, 1, _SC_H, _W), jnp.float32),
        mesh=mesh,
        scratch_types=[
            pltpu.VMEM((2, _DQ, _ROWS, _W), jnp.float32),
            pltpu.VMEM((4, _ROWS * _W), jnp.float32),
            pltpu.VMEM((_ROWS, _W), jnp.float32),
            pltpu.SemaphoreType.DMA,
            pltpu.SemaphoreType.DMA,
            pltpu.SemaphoreType.DMA,
        ],
    )(cost)

    disp_tc = pl.pallas_call(
        _tc_body,
        grid=(_B, _TC_H // _TC_BH),
        in_specs=[
            pl.BlockSpec((1, _D, _TC_BH, _W), lambda i, j: (i, 0, j, 0))
        ],
        out_specs=pl.BlockSpec((1, 1, _TC_BH, _W), lambda i, j: (i, 0, j, 0)),
        out_shape=jax.ShapeDtypeStruct((_B, 1, _H, _W), jnp.float32),
    )(cost)

    return lax.dynamic_update_slice(disp_tc, disp_sc, (0, 0, _TC_H, 0))
